# baseline (speedup 1.0000x reference)
import functools

import jax
import jax.numpy as jnp
from jax import lax
from jax.experimental import pallas as pl
from jax.experimental.pallas import tpu as pltpu
from jax.experimental.pallas import tpu_sc as plsc

_EMB = 64
_PATH = 20
_L = 16
_BLK = 128


def _log_f32(x):
    bits = lax.bitcast_convert_type(x, jnp.int32)
    e = (bits >> 23) - 127
    mbits = (bits & 0x007FFFFF) | 0x3F800000
    m = lax.bitcast_convert_type(mbits, jnp.float32)
    big = m > 1.4142135381698608
    m = jnp.where(big, m * 0.5, m)
    ef = e.astype(jnp.float32) + jnp.where(big, 1.0, 0.0)
    t = (m - 1.0) / (m + 1.0)
    t2 = t * t
    poly = 1.0 + t2 * (
        0.3333333432674408
        + t2 * (0.20000000298023224 + t2 * (0.1428571492433548 + t2 * 0.1111111119389534))
    )
    return ef * 0.6931471805599453 + 2.0 * t * poly


def _extract_col(blk, m_vec):
    return [
        plsc.load_gather(blk, [lax.iota(jnp.int32, 16) + 16 * j, m_vec])
        for j in range(_EMB // _L)
    ]


def _sc_body(in_t, node_t, idx_hbm, tgt_hbm, codes_hbm, out_hbm,
             idx_vtmp, codes_v, v_blk, blk1, blk2, row_v,
             sem_s, sem_u, sem_v):
    cid = lax.axis_index("c")
    sid = lax.axis_index("s")

    @pl.when(cid == 0)
    def _():
        st = [
            pltpu.make_async_copy(idx_hbm, idx_vtmp.at[pl.ds(0, _PATH)], sem_s),
            pltpu.make_async_copy(tgt_hbm, idx_vtmp.at[pl.ds(32, 1)], sem_s),
            pltpu.make_async_copy(codes_hbm, codes_v.at[pl.ds(0, _PATH)], sem_s),
        ]
        for cp in st:
            cp.start()
        for cp in st:
            cp.wait()

        lane = lax.iota(jnp.int32, 16)
        iv0 = idx_vtmp[pl.ds(0, _L)]
        iv1 = idx_vtmp[pl.ds(_L, _L)]
        cv0 = codes_v[pl.ds(0, _L)]
        cv1 = codes_v[pl.ds(_L, _L)]
        mine = lane == sid
        i1 = jnp.sum(jnp.where(mine, iv0, 0))
        has2 = sid < (_PATH - _L)
        i2 = jnp.where(has2, jnp.sum(jnp.where(mine, iv1, 0)), 0)
        cd1 = jnp.sum(jnp.where(mine, cv0, 0))
        cd2 = jnp.sum(jnp.where(mine, cv1, 0))
        tgt = jnp.sum(jnp.where(lane == 0, idx_vtmp[pl.ds(32, _L)], 0))

        def fetch(tbl, r, dst, sem):
            base = pl.multiple_of((r // _BLK) * _BLK, _BLK)
            cp = pltpu.make_async_copy(tbl.at[:, pl.ds(base, _BLK)], dst, sem)
            cp.start()
            return cp

        cps = [fetch(in_t, tgt, v_blk, sem_v),
               fetch(node_t, i1, blk1, sem_u)]

        @pl.when(has2)
        def _():
            fetch(node_t, i2, blk2, sem_u).wait()

        for cp in cps:
            cp.wait()

        vch = _extract_col(v_blk, jnp.broadcast_to(tgt & (_BLK - 1), (_L,)))

        def dot(blk, r):
            uch = _extract_col(blk, jnp.broadcast_to(r & (_BLK - 1), (_L,)))
            prod = uch[0] * vch[0]
            for j in range(1, _EMB // _L):
                prod = prod + uch[j] * vch[j]
            return jnp.sum(prod)

        d1 = dot(blk1, i1)
        d2 = dot(blk2, i2)

        zero = jnp.zeros((_L,), jnp.float32)
        dvec = jnp.where(lane == 0, jnp.broadcast_to(d1, (_L,)),
                         jnp.where(lane == 1, jnp.broadcast_to(d2, (_L,)), zero))
        cdvec = jnp.where(lane == 0, jnp.broadcast_to(cd1, (_L,)),
                          jnp.broadcast_to(cd2, (_L,)))
        z = jnp.where(cdvec == 1, dvec, -dvec)

        p = 1.0 / (1.0 + jnp.exp(-z))
        l = _log_f32(p + 1e-9)
        valid = jnp.logical_or(lane == 0, jnp.logical_and(lane == 1, has2))
        row_v[...] = jnp.where(valid, -l, 0.0)
        pltpu.sync_copy(row_v, out_hbm.at[sid])


@functools.cache
def _build_sc_fn():
  return pl.kernel(
    _sc_body,
    out_type=jax.ShapeDtypeStruct((_L, _L), jnp.float32),
    mesh=plsc.VectorSubcoreMesh(
        core_axis_name="c", subcore_axis_name="s", num_cores=1
    ),
    scratch_types=[
        pltpu.VMEM((48,), jnp.int32),
        pltpu.VMEM((32,), jnp.int32),
        pltpu.VMEM((_EMB, _BLK), jnp.float32),
        pltpu.VMEM((_EMB, _BLK), jnp.float32),
        pltpu.VMEM((_EMB, _BLK), jnp.float32),
        pltpu.VMEM((_L,), jnp.float32),
        pltpu.SemaphoreType.DMA,
        pltpu.SemaphoreType.DMA,
        pltpu.SemaphoreType.DMA,
    ],
    compiler_params=pltpu.CompilerParams(needs_layout_passes=False),
  )


@jax.jit
def kernel(in_table, node_table, target_idx, node_ids, codes):
    in_t = jnp.swapaxes(in_table, 0, 1)
    node_t = jnp.swapaxes(node_table, 0, 1)
    out = _build_sc_fn()(
        in_t, node_t,
        node_ids.astype(jnp.int32),
        target_idx.astype(jnp.int32).reshape(1),
        codes.astype(jnp.int32),
    )
    return jnp.sum(out)
